# baseline (speedup 1.0000x reference)
import functools

import jax
import jax.numpy as jnp
from jax import lax
from jax.experimental import pallas as pl
from jax.experimental.pallas import tpu as pltpu
from jax.experimental.pallas import tpu_sc as plsc


def _add_block(patches_ref, pos_ref, out_ref):
    out_ref[...] = patches_ref[...] + pos_ref[...]


def _tc_add(patches, pos_table, bb=8):
    batch, n_patches, model_dim = patches.shape
    return pl.pallas_call(
        _add_block,
        grid=(batch // bb,),
        compiler_params=pltpu.CompilerParams(vmem_limit_bytes=120 * 1024 * 1024),
        in_specs=[
            pl.BlockSpec((bb, n_patches, model_dim), lambda i: (i, 0, 0)),
            pl.BlockSpec((n_patches, model_dim), lambda i: (0, 0)),
        ],
        out_specs=pl.BlockSpec((bb, n_patches, model_dim), lambda i: (i, 0, 0)),
        out_shape=jax.ShapeDtypeStruct((batch, n_patches, model_dim), patches.dtype),
    )(patches, pos_table)


_NC = 2
_NS = 16
_L = 16
_NW = _NC * _NS

_BATCH = 64
_ROW = 576 * 768
_BPW = _BATCH // _NW
_CH = 64 * 768
_NCHUNK = _ROW // _CH
_UNROLL = 8


def _sc_body(pf_hbm, posf_hbm, out_hbm, pos_v, pat_v):
    wid = lax.axis_index("s") * _NC + lax.axis_index("c")

    def chunk_body(c, carry):
        pltpu.sync_copy(posf_hbm.at[pl.ds(c * _CH, _CH)], pos_v)

        def batch_body(bi, carry2):
            base = (wid * _BPW + bi) * _ROW + c * _CH
            pltpu.sync_copy(pf_hbm.at[pl.ds(base, _CH)], pat_v)

            def vec_body(i, carry3):
                off = i * (_L * _UNROLL)
                for k in range(_UNROLL):
                    o = off + k * _L
                    pat_v[pl.ds(o, _L)] = pat_v[pl.ds(o, _L)] + pos_v[pl.ds(o, _L)]
                return carry3

            lax.fori_loop(0, _CH // (_L * _UNROLL), vec_body, 0)
            pltpu.sync_copy(pat_v, out_hbm.at[pl.ds(base, _CH)])
            return carry2

        lax.fori_loop(0, _BPW, batch_body, 0)
        return carry

    lax.fori_loop(0, _NCHUNK, chunk_body, 0)


def _sc_add(patches_flat, pos_flat):
    mesh = plsc.VectorSubcoreMesh(core_axis_name="c", subcore_axis_name="s")
    f = functools.partial(
        pl.kernel,
        mesh=mesh,
        out_type=jax.ShapeDtypeStruct((_BATCH * _ROW,), jnp.float32),
        scratch_types=[
            pltpu.VMEM((_CH,), jnp.float32),
            pltpu.VMEM((_CH,), jnp.float32),
        ],
    )(_sc_body)
    return f(patches_flat, pos_flat)


def kernel(patches, pos_table):
    batch, n_patches, model_dim = patches.shape
    out_flat = _sc_add(patches.reshape(-1), pos_table.reshape(-1))
    return out_flat.reshape(batch, n_patches, model_dim)


# speedup vs baseline: 1.1964x; 1.1964x over previous
import functools

import jax
import jax.numpy as jnp
from jax import lax
from jax.experimental import pallas as pl
from jax.experimental.pallas import tpu as pltpu
from jax.experimental.pallas import tpu_sc as plsc


def _add_block(patches_ref, pos_ref, out_ref):
    out_ref[...] = patches_ref[...] + pos_ref[...]


def _tc_add(patches, pos_table, bb=8):
    batch, n_patches, model_dim = patches.shape
    return pl.pallas_call(
        _add_block,
        grid=(batch // bb,),
        compiler_params=pltpu.CompilerParams(vmem_limit_bytes=120 * 1024 * 1024),
        in_specs=[
            pl.BlockSpec((bb, n_patches, model_dim), lambda i: (i, 0, 0)),
            pl.BlockSpec((n_patches, model_dim), lambda i: (0, 0)),
        ],
        out_specs=pl.BlockSpec((bb, n_patches, model_dim), lambda i: (i, 0, 0)),
        out_shape=jax.ShapeDtypeStruct((batch, n_patches, model_dim), patches.dtype),
    )(patches, pos_table)


_NC = 2
_NS = 16
_L = 16
_NW = _NC * _NS

_BATCH = 64
_ROW = 576 * 768
_BPW = _BATCH // _NW
_CH = 24 * 768
_NCHUNK = _ROW // _CH
_NSTEP = _NCHUNK * _BPW
_UNROLL = 8


def _sc_body(pf_hbm, posf_hbm, out_hbm,
             pat0, pat1, pat2, pat3, posA, posB,
             pin0, pin1, pin2, pin3, pout0, pout1, pout2, pout3, psA, psB):
    wid = lax.axis_index("s") * _NC + lax.axis_index("c")
    pat = [pat0, pat1, pat2, pat3]
    pin = [pin0, pin1, pin2, pin3]
    pout = [pout0, pout1, pout2, pout3]
    pos = [posA, posB]
    pss = [psA, psB]

    def pat_base(t):
        c = t // 2
        bi = t - 2 * c if not isinstance(t, int) else t % 2
        return (wid * _BPW + bi) * _ROW + c * _CH

    def start_pat_in(t, j):
        pltpu.async_copy(pf_hbm.at[pl.ds(pat_base(t), _CH)], pat[j], pin[j])

    def wait_pat_in(j):
        pltpu.make_async_copy(pf_hbm.at[pl.ds(0, _CH)], pat[j], pin[j]).wait()

    def start_pat_out(t, j):
        pltpu.async_copy(pat[j], out_hbm.at[pl.ds(pat_base(t), _CH)], pout[j])

    def wait_pat_out(j):
        pltpu.make_async_copy(pat[j], out_hbm.at[pl.ds(0, _CH)], pout[j]).wait()

    def start_pos(c, k):
        c = jnp.minimum(c, _NCHUNK - 1)
        pltpu.async_copy(posf_hbm.at[pl.ds(c * _CH, _CH)], pos[k], pss[k])

    def wait_pos(k):
        pltpu.make_async_copy(posf_hbm.at[pl.ds(0, _CH)], pos[k], pss[k]).wait()

    def compute(j, k):
        def vec_body(i, carry):
            off = i * (_L * _UNROLL)
            for u in range(_UNROLL):
                o = off + u * _L
                plsc.addupdate(pat[j].at[pl.ds(o, _L)], pos[k][pl.ds(o, _L)])
            return carry

        lax.fori_loop(0, _CH // (_L * _UNROLL), vec_body, 0)

    def do_step(t, tstat, first_block, last_block):
        u = tstat % 4
        j = u
        k = (tstat // 2) % 2
        if u == 0:
            wait_pos(k)
            start_pos(t // 2 + 1, 1 - k)
        if u == 2:
            wait_pos(k)
            if not last_block:
                start_pos(t // 2 + 1, 1 - k)
        wait_pat_in(j)
        compute(j, k)
        start_pat_out(t, j)
        if not last_block or u == 0:
            jf = (u + 3) % 4
            if not (first_block and u == 0):
                wait_pat_out(jf)
            start_pat_in(t + 3, jf)

    start_pos(0, 0)
    for s in range(3):
        start_pat_in(s, s)
    for u in range(4):
        do_step(u, u, first_block=True, last_block=False)

    def block_body(m, carry):
        for u in range(4):
            do_step(4 * m + u, u, first_block=False, last_block=False)
        return carry

    lax.fori_loop(1, _NSTEP // 4 - 1, block_body, 0)

    mlast = _NSTEP // 4 - 1
    for u in range(4):
        do_step(4 * mlast + u, u, first_block=False, last_block=True)

    for j in range(4):
        wait_pat_out(j)


def _sc_add(patches_flat, pos_flat):
    mesh = plsc.VectorSubcoreMesh(core_axis_name="c", subcore_axis_name="s")
    f = functools.partial(
        pl.kernel,
        mesh=mesh,
        out_type=jax.ShapeDtypeStruct((_BATCH * _ROW,), jnp.float32),
        scratch_types=(
            [pltpu.VMEM((_CH,), jnp.float32)] * 6
            + [pltpu.SemaphoreType.DMA] * 10
        ),
    )(_sc_body)
    return f(patches_flat, pos_flat)


def kernel(patches, pos_table):
    batch, n_patches, model_dim = patches.shape
    out_flat = _sc_add(patches.reshape(-1), pos_table.reshape(-1))
    return out_flat.reshape(batch, n_patches, model_dim)


# speedup vs baseline: 6.3152x; 5.2785x over previous
import functools

import jax
import jax.numpy as jnp
from jax import lax
from jax.experimental import pallas as pl
from jax.experimental.pallas import tpu as pltpu
from jax.experimental.pallas import tpu_sc as plsc


def _add_block(patches_ref, pos_ref, out_ref):
    out_ref[...] = patches_ref[...] + pos_ref[...]


def _tc_add(patches, pos_table, bb=8):
    batch, n_patches, model_dim = patches.shape
    return pl.pallas_call(
        _add_block,
        grid=(batch // bb,),
        compiler_params=pltpu.CompilerParams(vmem_limit_bytes=120 * 1024 * 1024),
        in_specs=[
            pl.BlockSpec((bb, n_patches, model_dim), lambda i: (i, 0, 0)),
            pl.BlockSpec((n_patches, model_dim), lambda i: (0, 0)),
        ],
        out_specs=pl.BlockSpec((bb, n_patches, model_dim), lambda i: (i, 0, 0)),
        out_shape=jax.ShapeDtypeStruct((batch, n_patches, model_dim), patches.dtype),
    )(patches, pos_table)


_NC = 2
_NS = 16
_L = 16
_NW = _NC * _NS

_BATCH = 64
_ROW = 576 * 768
_BPW = _BATCH // _NW
_CH = 24 * 768
_NCHUNK = _ROW // _CH
_NSTEP = _NCHUNK * _BPW
_UNROLL = 8


def _sc_body(pf_hbm, posf_hbm, out_hbm,
             pat0, pat1, pat2, pat3, posA, posB,
             pin0, pin1, pin2, pin3, pout0, pout1, pout2, pout3, psA, psB):
    wid = lax.axis_index("s") * _NC + lax.axis_index("c")
    pat = [pat0, pat1, pat2, pat3]
    pin = [pin0, pin1, pin2, pin3]
    pout = [pout0, pout1, pout2, pout3]
    pos = [posA, posB]
    pss = [psA, psB]

    def pat_base(t):
        c = t // 2
        bi = t - 2 * c if not isinstance(t, int) else t % 2
        return (wid * _BPW + bi) * _ROW + c * _CH

    def start_pat_in(t, j):
        pltpu.async_copy(pf_hbm.at[pl.ds(pat_base(t), _CH)], pat[j], pin[j])

    def wait_pat_in(j):
        pltpu.make_async_copy(pf_hbm.at[pl.ds(0, _CH)], pat[j], pin[j]).wait()

    def start_pat_out(t, j):
        pltpu.async_copy(pat[j], out_hbm.at[pl.ds(pat_base(t), _CH)], pout[j])

    def wait_pat_out(j):
        pltpu.make_async_copy(pat[j], out_hbm.at[pl.ds(0, _CH)], pout[j]).wait()

    def start_pos(c, k):
        c = jnp.minimum(c, _NCHUNK - 1)
        pltpu.async_copy(posf_hbm.at[pl.ds(c * _CH, _CH)], pos[k], pss[k])

    def wait_pos(k):
        pltpu.make_async_copy(posf_hbm.at[pl.ds(0, _CH)], pos[k], pss[k]).wait()

    def compute(j, k):
        def vec_body(i, carry):
            off = i * (_L * _UNROLL)
            for u in range(_UNROLL):
                o = off + u * _L
                plsc.addupdate(pat[j].at[pl.ds(o, _L)], pos[k][pl.ds(o, _L)])
            return carry

        lax.fori_loop(0, _CH // (_L * _UNROLL), vec_body, 0)

    def do_step(t, tstat, first_block, last_block):
        u = tstat % 4
        j = u
        k = (tstat // 2) % 2
        if u == 0:
            wait_pos(k)
            start_pos(t // 2 + 1, 1 - k)
        if u == 2:
            wait_pos(k)
            if not last_block:
                start_pos(t // 2 + 1, 1 - k)
        wait_pat_in(j)
        compute(j, k)
        start_pat_out(t, j)
        if not last_block or u == 0:
            jf = (u + 3) % 4
            if not (first_block and u == 0):
                wait_pat_out(jf)
            start_pat_in(t + 3, jf)

    start_pos(0, 0)
    for s in range(3):
        start_pat_in(s, s)
    for u in range(4):
        do_step(u, u, first_block=True, last_block=False)

    def block_body(m, carry):
        for u in range(4):
            do_step(4 * m + u, u, first_block=False, last_block=False)
        return carry

    lax.fori_loop(1, _NSTEP // 4 - 1, block_body, 0)

    mlast = _NSTEP // 4 - 1
    for u in range(4):
        do_step(4 * mlast + u, u, first_block=False, last_block=True)

    for j in range(4):
        wait_pat_out(j)


def _sc_add(patches_flat, pos_flat):
    mesh = plsc.VectorSubcoreMesh(core_axis_name="c", subcore_axis_name="s")
    f = functools.partial(
        pl.kernel,
        mesh=mesh,
        out_type=jax.ShapeDtypeStruct((_BATCH * _ROW,), jnp.float32),
        scratch_types=(
            [pltpu.VMEM((_CH,), jnp.float32)] * 6
            + [pltpu.SemaphoreType.DMA] * 10
        ),
    )(_sc_body)
    return f(patches_flat, pos_flat)


def kernel(patches, pos_table):
    return _tc_add(patches, pos_table, bb=8)
